# baseline (speedup 1.0000x reference)
import functools

import jax
import jax.numpy as jnp
from jax import lax
from jax.experimental import pallas as pl
from jax.experimental.pallas import tpu as pltpu
from jax.experimental.pallas import tpu_sc as plsc

N = 10000
E = 320000
F_IN = 128
DIM = 32
NCLS = 10
NG = 64

NCORE = 2
NSUB = 16
NW = NCORE * NSUB
CHUNK = 128
NPAD = 10112
RPT = NPAD // NSUB

INV_STD = 1.0 / (1.0 + 1e-5) ** 0.5


SUBK = 8
BLKE = SUBK * CHUNK
NBLK = -(-E // (NW * BLKE))
NCHUNK = NBLK * SUBK
EPW = NBLK * BLKE
EPAD = NW * EPW


def _edge_sum_body(zeros_hbm, table_hbm, srcr_hbm, dstr_hbm, out_hbm,
                   src_v, dst_v, rows0_v, rows1_v, acc_sh, table_sh,
                   gsem0, gsem1):
    cid = lax.axis_index("c")
    sid = lax.axis_index("s")
    w = cid * NSUB + sid
    pltpu.sync_copy(zeros_hbm.at[pl.ds(sid * RPT, RPT)],
                    acc_sh.at[pl.ds(sid * RPT, RPT)])
    pltpu.sync_copy(table_hbm.at[pl.ds(sid * RPT, RPT)],
                    table_sh.at[pl.ds(sid * RPT, RPT)])
    pltpu.sync_copy(srcr_hbm.at[w], src_v)
    pltpu.sync_copy(dstr_hbm.at[w], dst_v)
    plsc.subcore_barrier()

    bufs = (rows0_v, rows1_v)
    sems = (gsem0, gsem1)

    def _gather(j, b):
        return pltpu.async_copy(
            table_sh.at[src_v.at[pl.ds(j * BLKE, BLKE)]], bufs[b], sems[b])

    g = _gather(0, 0)
    for j in range(NBLK):
        g.wait()
        if j + 1 < NBLK:
            g = _gather(j + 1, (j + 1) % 2)
        for b in range(SUBK):
            pltpu.sync_copy(bufs[j % 2].at[pl.ds(b * CHUNK, CHUNK)],
                            acc_sh.at[dst_v.at[j * SUBK + b]], add=True)

    plsc.subcore_barrier()
    pltpu.sync_copy(acc_sh.at[pl.ds(sid * RPT, RPT)],
                    out_hbm.at[cid, pl.ds(sid * RPT, RPT)])


_edge_sum = pl.kernel(
    _edge_sum_body,
    out_type=jax.ShapeDtypeStruct((NCORE, NPAD, DIM), jnp.float32),
    mesh=plsc.VectorSubcoreMesh(core_axis_name="c", subcore_axis_name="s"),
    scratch_types=[
        pltpu.VMEM((EPW,), jnp.int32),
        pltpu.VMEM((NCHUNK, CHUNK), jnp.int32),
        pltpu.VMEM((BLKE, DIM), jnp.float32),
        pltpu.VMEM((BLKE, DIM), jnp.float32),
        pltpu.VMEM_SHARED((NPAD, DIM), jnp.float32),
        pltpu.VMEM_SHARED((NPAD, DIM), jnp.float32),
        pltpu.SemaphoreType.DMA,
        pltpu.SemaphoreType.DMA,
    ],
    compiler_params=pltpu.CompilerParams(use_tc_tiling_on_sc=False),
)


def _proj_body(x_ref, w_ref, o_ref):
    o_ref[...] = jnp.dot(x_ref[...], w_ref[...],
                         preferred_element_type=jnp.float32)


def _proj(x, w):
    return pl.pallas_call(
        _proj_body,
        out_shape=jax.ShapeDtypeStruct((NPAD, DIM), jnp.float32),
    )(x, w)


def _mlp_body(a_ref, z_ref, wa_ref, ba_ref, wb_ref, bb_ref, g_ref, bt_ref,
              o_ref):
    agg = z_ref[...] + a_ref[0] + a_ref[1]
    t = jnp.dot(agg, wa_ref[...], preferred_element_type=jnp.float32)
    t = jnp.maximum(t + ba_ref[...], 0.0)
    t = jnp.dot(t, wb_ref[...], preferred_element_type=jnp.float32)
    t = jnp.maximum(t + bb_ref[...], 0.0)
    o_ref[...] = g_ref[...] * (t * INV_STD) + bt_ref[...]


def _mlp(a, z, wa, ba, wb, bb, g, bt):
    return pl.pallas_call(
        _mlp_body,
        out_shape=jax.ShapeDtypeStruct((NPAD, DIM), jnp.float32),
    )(a, z, wa, ba, wb, bb, g, bt)


def _head_body(h_ref, b_ref, w1_ref, b1_ref, w2_ref, b2_ref, o_ref):
    gids = lax.broadcasted_iota(jnp.int32, (NG, NPAD), 0)
    m = (gids == b_ref[...]).astype(jnp.float32)
    pooled = jnp.dot(m, h_ref[...], preferred_element_type=jnp.float32)
    t = jnp.dot(pooled, w1_ref[...], preferred_element_type=jnp.float32)
    t = jnp.maximum(t + b1_ref[...], 0.0)
    logits = jnp.dot(t, w2_ref[...], preferred_element_type=jnp.float32)
    logits = logits + b2_ref[...]
    mx = jnp.max(logits, axis=-1, keepdims=True)
    lse = jnp.log(jnp.sum(jnp.exp(logits - mx), axis=-1, keepdims=True)) + mx
    o_ref[...] = logits - lse


def _head(h, b, w1, b1, w2, b2):
    return pl.pallas_call(
        _head_body,
        out_shape=jax.ShapeDtypeStruct((NG, NCLS), jnp.float32),
    )(h, b, w1, b1, w2, b2)


def kernel(x, edge_index, batch, w1a, b1a, w1b, b1b, Wa, Ba, Wb, Bb,
           gamma, beta, fc1_w, fc1_b, fc2_w, fc2_b):
    pad_e = EPAD - E
    pad_idx = N + (jnp.arange(pad_e, dtype=jnp.int32) % (NPAD - N))
    src = jnp.concatenate([edge_index[0], pad_idx]).reshape(NW, EPW)
    dst = jnp.concatenate([edge_index[1], pad_idx]).reshape(
        NW, NCHUNK, CHUNK)
    xp = jnp.pad(x, ((0, NPAD - N), (0, 0)))
    zeros = jnp.zeros((NPAD, DIM), jnp.float32)
    eye = jnp.eye(DIM, dtype=jnp.float32)
    bpad = jnp.pad(batch, (0, NPAD - N), constant_values=NG).reshape(1, NPAD)

    was = [eye] + [Wa[i] for i in range(4)]
    bas = [b1a.reshape(1, DIM)] + [Ba[i].reshape(1, DIM) for i in range(4)]
    wbs = [w1b] + [Wb[i] for i in range(4)]
    bbs = [b1b.reshape(1, DIM)] + [Bb[i].reshape(1, DIM) for i in range(4)]

    z = _proj(xp, w1a)
    for i in range(5):
        a = _edge_sum(zeros, z, src, dst)
        z = _mlp(a, z, was[i], bas[i], wbs[i], bbs[i],
                 gamma[i].reshape(1, DIM), beta[i].reshape(1, DIM))
    return _head(z, bpad, fc1_w, fc1_b.reshape(1, DIM),
                 fc2_w, fc2_b.reshape(1, NCLS))


# speedup vs baseline: 1.0892x; 1.0892x over previous
import functools

import jax
import jax.numpy as jnp
from jax import lax
from jax.experimental import pallas as pl
from jax.experimental.pallas import tpu as pltpu
from jax.experimental.pallas import tpu_sc as plsc

N = 10000
E = 320000
F_IN = 128
DIM = 32
NCLS = 10
NG = 64

NCORE = 2
NSUB = 16
NW = NCORE * NSUB
CHUNK = 256
NPAD = 10112
RPT = NPAD // NSUB

INV_STD = 1.0 / (1.0 + 1e-5) ** 0.5


SUBK = 4
BLKE = SUBK * CHUNK
NBLK = -(-E // (NW * BLKE))
NCHUNK = NBLK * SUBK
EPW = NBLK * BLKE
EPAD = NW * EPW


def _edge_sum_body(zeros_hbm, table_hbm, srcr_hbm, dstr_hbm, out_hbm,
                   src_v, dst_v, rows0_v, rows1_v, acc_sh, gsem0, gsem1):
    cid = lax.axis_index("c")
    sid = lax.axis_index("s")
    w = cid * NSUB + sid
    pltpu.sync_copy(zeros_hbm.at[pl.ds(sid * RPT, RPT)],
                    acc_sh.at[pl.ds(sid * RPT, RPT)])
    pltpu.sync_copy(srcr_hbm.at[w], src_v)
    pltpu.sync_copy(dstr_hbm.at[w], dst_v)
    plsc.subcore_barrier()

    bufs = (rows0_v, rows1_v)
    sems = (gsem0, gsem1)

    def _gather(j, b):
        return pltpu.async_copy(
            table_hbm.at[src_v.at[pl.ds(j * BLKE, BLKE)]], bufs[b], sems[b])

    g = _gather(0, 0)
    for j in range(NBLK):
        g.wait()
        if j + 1 < NBLK:
            g = _gather(j + 1, (j + 1) % 2)
        for b in range(SUBK):
            pltpu.sync_copy(bufs[j % 2].at[pl.ds(b * CHUNK, CHUNK)],
                            acc_sh.at[dst_v.at[j * SUBK + b]], add=True)

    plsc.subcore_barrier()
    pltpu.sync_copy(acc_sh.at[pl.ds(sid * RPT, RPT)],
                    out_hbm.at[cid, pl.ds(sid * RPT, RPT)])


_edge_sum = pl.kernel(
    _edge_sum_body,
    out_type=jax.ShapeDtypeStruct((NCORE, NPAD, DIM), jnp.float32),
    mesh=plsc.VectorSubcoreMesh(core_axis_name="c", subcore_axis_name="s"),
    scratch_types=[
        pltpu.VMEM((EPW,), jnp.int32),
        pltpu.VMEM((NCHUNK, CHUNK), jnp.int32),
        pltpu.VMEM((BLKE, DIM), jnp.float32),
        pltpu.VMEM((BLKE, DIM), jnp.float32),
        pltpu.VMEM_SHARED((NPAD, DIM), jnp.float32),
        pltpu.SemaphoreType.DMA,
        pltpu.SemaphoreType.DMA,
    ],
    compiler_params=pltpu.CompilerParams(use_tc_tiling_on_sc=False),
)


def _proj_body(x_ref, w_ref, o_ref):
    o_ref[...] = jnp.dot(x_ref[...], w_ref[...],
                         preferred_element_type=jnp.float32)


def _proj(x, w):
    return pl.pallas_call(
        _proj_body,
        out_shape=jax.ShapeDtypeStruct((NPAD, DIM), jnp.float32),
    )(x, w)


def _mlp_body(a_ref, z_ref, wa_ref, ba_ref, wb_ref, bb_ref, g_ref, bt_ref,
              o_ref):
    agg = z_ref[...] + a_ref[0] + a_ref[1]
    t = jnp.dot(agg, wa_ref[...], preferred_element_type=jnp.float32)
    t = jnp.maximum(t + ba_ref[...], 0.0)
    t = jnp.dot(t, wb_ref[...], preferred_element_type=jnp.float32)
    t = jnp.maximum(t + bb_ref[...], 0.0)
    o_ref[...] = g_ref[...] * (t * INV_STD) + bt_ref[...]


def _mlp(a, z, wa, ba, wb, bb, g, bt):
    return pl.pallas_call(
        _mlp_body,
        out_shape=jax.ShapeDtypeStruct((NPAD, DIM), jnp.float32),
    )(a, z, wa, ba, wb, bb, g, bt)


def _head_body(h_ref, b_ref, w1_ref, b1_ref, w2_ref, b2_ref, o_ref):
    gids = lax.broadcasted_iota(jnp.int32, (NG, NPAD), 0)
    m = (gids == b_ref[...]).astype(jnp.float32)
    pooled = jnp.dot(m, h_ref[...], preferred_element_type=jnp.float32)
    t = jnp.dot(pooled, w1_ref[...], preferred_element_type=jnp.float32)
    t = jnp.maximum(t + b1_ref[...], 0.0)
    logits = jnp.dot(t, w2_ref[...], preferred_element_type=jnp.float32)
    logits = logits + b2_ref[...]
    mx = jnp.max(logits, axis=-1, keepdims=True)
    lse = jnp.log(jnp.sum(jnp.exp(logits - mx), axis=-1, keepdims=True)) + mx
    o_ref[...] = logits - lse


def _head(h, b, w1, b1, w2, b2):
    return pl.pallas_call(
        _head_body,
        out_shape=jax.ShapeDtypeStruct((NG, NCLS), jnp.float32),
    )(h, b, w1, b1, w2, b2)


def kernel(x, edge_index, batch, w1a, b1a, w1b, b1b, Wa, Ba, Wb, Bb,
           gamma, beta, fc1_w, fc1_b, fc2_w, fc2_b):
    pad_e = EPAD - E
    pad_idx = N + (jnp.arange(pad_e, dtype=jnp.int32) % (NPAD - N))
    src = jnp.concatenate([edge_index[0], pad_idx]).reshape(NW, EPW)
    dst = jnp.concatenate([edge_index[1], pad_idx]).reshape(
        NW, NCHUNK, CHUNK)
    xp = jnp.pad(x, ((0, NPAD - N), (0, 0)))
    zeros = jnp.zeros((NPAD, DIM), jnp.float32)
    eye = jnp.eye(DIM, dtype=jnp.float32)
    bpad = jnp.pad(batch, (0, NPAD - N), constant_values=NG).reshape(1, NPAD)

    was = [eye] + [Wa[i] for i in range(4)]
    bas = [b1a.reshape(1, DIM)] + [Ba[i].reshape(1, DIM) for i in range(4)]
    wbs = [w1b] + [Wb[i] for i in range(4)]
    bbs = [b1b.reshape(1, DIM)] + [Bb[i].reshape(1, DIM) for i in range(4)]

    z = _proj(xp, w1a)
    for i in range(5):
        a = _edge_sum(zeros, z, src, dst)
        z = _mlp(a, z, was[i], bas[i], wbs[i], bbs[i],
                 gamma[i].reshape(1, DIM), beta[i].reshape(1, DIM))
    return _head(z, bpad, fc1_w, fc1_b.reshape(1, DIM),
                 fc2_w, fc2_b.reshape(1, NCLS))


# speedup vs baseline: 1.0968x; 1.0070x over previous
import functools

import jax
import jax.numpy as jnp
from jax import lax
from jax.experimental import pallas as pl
from jax.experimental.pallas import tpu as pltpu
from jax.experimental.pallas import tpu_sc as plsc

N = 10000
E = 320000
F_IN = 128
DIM = 32
NCLS = 10
NG = 64

NCORE = 2
NSUB = 16
NW = NCORE * NSUB
CHUNK = 512
NPAD = 10112
RPT = NPAD // NSUB

INV_STD = 1.0 / (1.0 + 1e-5) ** 0.5


SUBK = 2
BLKE = SUBK * CHUNK
NBLK = -(-E // (NW * BLKE))
NCHUNK = NBLK * SUBK
EPW = NBLK * BLKE
EPAD = NW * EPW


def _edge_sum_body(zeros_hbm, table_hbm, srcr_hbm, dstr_hbm, out_hbm,
                   src_v, dst_v, rows0_v, rows1_v, acc_sh, gsem0, gsem1):
    cid = lax.axis_index("c")
    sid = lax.axis_index("s")
    w = cid * NSUB + sid
    pltpu.sync_copy(zeros_hbm.at[pl.ds(sid * RPT, RPT)],
                    acc_sh.at[pl.ds(sid * RPT, RPT)])
    pltpu.sync_copy(srcr_hbm.at[w], src_v)
    pltpu.sync_copy(dstr_hbm.at[w], dst_v)
    plsc.subcore_barrier()

    bufs = (rows0_v, rows1_v)
    sems = (gsem0, gsem1)

    def _gather(j, b):
        return pltpu.async_copy(
            table_hbm.at[src_v.at[pl.ds(j * BLKE, BLKE)]], bufs[b], sems[b])

    g = _gather(0, 0)
    for j in range(NBLK):
        g.wait()
        if j + 1 < NBLK:
            g = _gather(j + 1, (j + 1) % 2)
        for b in range(SUBK):
            pltpu.sync_copy(bufs[j % 2].at[pl.ds(b * CHUNK, CHUNK)],
                            acc_sh.at[dst_v.at[j * SUBK + b]], add=True)

    plsc.subcore_barrier()
    pltpu.sync_copy(acc_sh.at[pl.ds(sid * RPT, RPT)],
                    out_hbm.at[cid, pl.ds(sid * RPT, RPT)])


_edge_sum = pl.kernel(
    _edge_sum_body,
    out_type=jax.ShapeDtypeStruct((NCORE, NPAD, DIM), jnp.float32),
    mesh=plsc.VectorSubcoreMesh(core_axis_name="c", subcore_axis_name="s"),
    scratch_types=[
        pltpu.VMEM((EPW,), jnp.int32),
        pltpu.VMEM((NCHUNK, CHUNK), jnp.int32),
        pltpu.VMEM((BLKE, DIM), jnp.float32),
        pltpu.VMEM((BLKE, DIM), jnp.float32),
        pltpu.VMEM_SHARED((NPAD, DIM), jnp.float32),
        pltpu.SemaphoreType.DMA,
        pltpu.SemaphoreType.DMA,
    ],
    compiler_params=pltpu.CompilerParams(use_tc_tiling_on_sc=False),
)


def _proj_body(x_ref, w_ref, o_ref):
    o_ref[...] = jnp.dot(x_ref[...], w_ref[...],
                         preferred_element_type=jnp.float32)


def _proj(x, w):
    return pl.pallas_call(
        _proj_body,
        out_shape=jax.ShapeDtypeStruct((NPAD, DIM), jnp.float32),
    )(x, w)


def _mlp_body(a_ref, z_ref, wa_ref, ba_ref, wb_ref, bb_ref, g_ref, bt_ref,
              o_ref):
    agg = z_ref[...] + a_ref[0] + a_ref[1]
    t = jnp.dot(agg, wa_ref[...], preferred_element_type=jnp.float32)
    t = jnp.maximum(t + ba_ref[...], 0.0)
    t = jnp.dot(t, wb_ref[...], preferred_element_type=jnp.float32)
    t = jnp.maximum(t + bb_ref[...], 0.0)
    o_ref[...] = g_ref[...] * (t * INV_STD) + bt_ref[...]


def _mlp(a, z, wa, ba, wb, bb, g, bt):
    return pl.pallas_call(
        _mlp_body,
        out_shape=jax.ShapeDtypeStruct((NPAD, DIM), jnp.float32),
    )(a, z, wa, ba, wb, bb, g, bt)


def _head_body(h_ref, b_ref, w1_ref, b1_ref, w2_ref, b2_ref, o_ref):
    gids = lax.broadcasted_iota(jnp.int32, (NG, NPAD), 0)
    m = (gids == b_ref[...]).astype(jnp.float32)
    pooled = jnp.dot(m, h_ref[...], preferred_element_type=jnp.float32)
    t = jnp.dot(pooled, w1_ref[...], preferred_element_type=jnp.float32)
    t = jnp.maximum(t + b1_ref[...], 0.0)
    logits = jnp.dot(t, w2_ref[...], preferred_element_type=jnp.float32)
    logits = logits + b2_ref[...]
    mx = jnp.max(logits, axis=-1, keepdims=True)
    lse = jnp.log(jnp.sum(jnp.exp(logits - mx), axis=-1, keepdims=True)) + mx
    o_ref[...] = logits - lse


def _head(h, b, w1, b1, w2, b2):
    return pl.pallas_call(
        _head_body,
        out_shape=jax.ShapeDtypeStruct((NG, NCLS), jnp.float32),
    )(h, b, w1, b1, w2, b2)


def kernel(x, edge_index, batch, w1a, b1a, w1b, b1b, Wa, Ba, Wb, Bb,
           gamma, beta, fc1_w, fc1_b, fc2_w, fc2_b):
    pad_e = EPAD - E
    pad_idx = N + (jnp.arange(pad_e, dtype=jnp.int32) % (NPAD - N))
    src = jnp.concatenate([edge_index[0], pad_idx]).reshape(NW, EPW)
    dst = jnp.concatenate([edge_index[1], pad_idx]).reshape(
        NW, NCHUNK, CHUNK)
    xp = jnp.pad(x, ((0, NPAD - N), (0, 0)))
    zeros = jnp.zeros((NPAD, DIM), jnp.float32)
    eye = jnp.eye(DIM, dtype=jnp.float32)
    bpad = jnp.pad(batch, (0, NPAD - N), constant_values=NG).reshape(1, NPAD)

    was = [eye] + [Wa[i] for i in range(4)]
    bas = [b1a.reshape(1, DIM)] + [Ba[i].reshape(1, DIM) for i in range(4)]
    wbs = [w1b] + [Wb[i] for i in range(4)]
    bbs = [b1b.reshape(1, DIM)] + [Bb[i].reshape(1, DIM) for i in range(4)]

    z = _proj(xp, w1a)
    for i in range(5):
        a = _edge_sum(zeros, z, src, dst)
        z = _mlp(a, z, was[i], bas[i], wbs[i], bbs[i],
                 gamma[i].reshape(1, DIM), beta[i].reshape(1, DIM))
    return _head(z, bpad, fc1_w, fc1_b.reshape(1, DIM),
                 fc2_w, fc2_b.reshape(1, NCLS))


# speedup vs baseline: 1.1123x; 1.0141x over previous
import functools

import jax
import jax.numpy as jnp
from jax import lax
from jax.experimental import pallas as pl
from jax.experimental.pallas import tpu as pltpu
from jax.experimental.pallas import tpu_sc as plsc

N = 10000
E = 320000
F_IN = 128
DIM = 32
NCLS = 10
NG = 64

NCORE = 2
NSUB = 16
NW = NCORE * NSUB
CHUNK = 512
NPAD = 10112
RPT = NPAD // NSUB

INV_STD = 1.0 / (1.0 + 1e-5) ** 0.5


SUBK = 2
BLKE = SUBK * CHUNK
NBLK = -(-E // (NW * BLKE))
NCHUNK = NBLK * SUBK
EPW = NBLK * BLKE
EPAD = NW * EPW


def _edge_sum_body(zeros_hbm, table_hbm, srcr_hbm, dstr_hbm, out_hbm,
                   src_v, dst_v, rows0_v, rows1_v, acc_sh, gsem0, gsem1):
    cid = lax.axis_index("c")
    sid = lax.axis_index("s")
    w = cid * NSUB + sid
    pltpu.sync_copy(zeros_hbm.at[pl.ds(sid * RPT, RPT)],
                    acc_sh.at[pl.ds(sid * RPT, RPT)])
    pltpu.sync_copy(srcr_hbm.at[w], src_v)
    pltpu.sync_copy(dstr_hbm.at[w], dst_v)
    plsc.subcore_barrier()

    bufs = (rows0_v, rows1_v)
    sems = (gsem0, gsem1)

    def _gather(j, b):
        return pltpu.async_copy(
            table_hbm.at[src_v.at[pl.ds(j * BLKE, BLKE)]], bufs[b], sems[b])

    g = _gather(0, 0)
    for j in range(NBLK):
        g.wait()
        if j + 1 < NBLK:
            g = _gather(j + 1, (j + 1) % 2)
        for b in range(SUBK):
            pltpu.sync_copy(bufs[j % 2].at[pl.ds(b * CHUNK, CHUNK)],
                            acc_sh.at[dst_v.at[j * SUBK + b]], add=True)

    plsc.subcore_barrier()
    pltpu.sync_copy(acc_sh.at[pl.ds(sid * RPT, RPT)],
                    out_hbm.at[cid, pl.ds(sid * RPT, RPT)])


_edge_sum = pl.kernel(
    _edge_sum_body,
    out_type=jax.ShapeDtypeStruct((NCORE, NPAD, DIM), jnp.float32),
    mesh=plsc.VectorSubcoreMesh(core_axis_name="c", subcore_axis_name="s"),
    scratch_types=[
        pltpu.VMEM((EPW,), jnp.int32),
        pltpu.VMEM((NCHUNK, CHUNK), jnp.int32),
        pltpu.VMEM((BLKE, DIM), jnp.float32),
        pltpu.VMEM((BLKE, DIM), jnp.float32),
        pltpu.VMEM_SHARED((NPAD, DIM), jnp.float32),
        pltpu.SemaphoreType.DMA,
        pltpu.SemaphoreType.DMA,
    ],
    compiler_params=pltpu.CompilerParams(use_tc_tiling_on_sc=False),
)


def _proj_body(x_ref, w_ref, o_ref):
    o_ref[...] = jnp.dot(x_ref[...], w_ref[...],
                         preferred_element_type=jnp.float32)


def _proj(x, w):
    return pl.pallas_call(
        _proj_body,
        out_shape=jax.ShapeDtypeStruct((NPAD, DIM), jnp.float32),
    )(x, w)


def _mlp_body(a_ref, z_ref, wa_ref, ba_ref, wb_ref, bb_ref, g_ref, bt_ref,
              o_ref):
    agg = z_ref[...] + a_ref[0] + a_ref[1]
    t = jnp.dot(agg, wa_ref[...], preferred_element_type=jnp.float32)
    t = jnp.maximum(t + ba_ref[...], 0.0)
    t = jnp.dot(t, wb_ref[...], preferred_element_type=jnp.float32)
    t = jnp.maximum(t + bb_ref[...], 0.0)
    o_ref[...] = g_ref[...] * (t * INV_STD) + bt_ref[...]


def _mlp(a, z, wa, ba, wb, bb, g, bt):
    return pl.pallas_call(
        _mlp_body,
        out_shape=jax.ShapeDtypeStruct((NPAD, DIM), jnp.float32),
    )(a, z, wa, ba, wb, bb, g, bt)


def _head_body(a_ref, z_ref, wa_ref, ba_ref, wb_ref, bb_ref, g_ref, bt_ref,
               b_ref, w1_ref, b1_ref, w2_ref, b2_ref, o_ref):
    agg = z_ref[...] + a_ref[0] + a_ref[1]
    t = jnp.dot(agg, wa_ref[...], preferred_element_type=jnp.float32)
    t = jnp.maximum(t + ba_ref[...], 0.0)
    t = jnp.dot(t, wb_ref[...], preferred_element_type=jnp.float32)
    t = jnp.maximum(t + bb_ref[...], 0.0)
    h = g_ref[...] * (t * INV_STD) + bt_ref[...]
    gids = lax.broadcasted_iota(jnp.int32, (NG, NPAD), 0)
    m = (gids == b_ref[...]).astype(jnp.float32)
    pooled = jnp.dot(m, h, preferred_element_type=jnp.float32)
    t = jnp.dot(pooled, w1_ref[...], preferred_element_type=jnp.float32)
    t = jnp.maximum(t + b1_ref[...], 0.0)
    logits = jnp.dot(t, w2_ref[...], preferred_element_type=jnp.float32)
    logits = logits + b2_ref[...]
    mx = jnp.max(logits, axis=-1, keepdims=True)
    lse = jnp.log(jnp.sum(jnp.exp(logits - mx), axis=-1, keepdims=True)) + mx
    o_ref[...] = logits - lse


def _head(a, z, wa, ba, wb, bb, g, bt, b, w1, b1, w2, b2):
    return pl.pallas_call(
        _head_body,
        out_shape=jax.ShapeDtypeStruct((NG, NCLS), jnp.float32),
    )(a, z, wa, ba, wb, bb, g, bt, b, w1, b1, w2, b2)


def kernel(x, edge_index, batch, w1a, b1a, w1b, b1b, Wa, Ba, Wb, Bb,
           gamma, beta, fc1_w, fc1_b, fc2_w, fc2_b):
    pad_e = EPAD - E
    pad_idx = N + (jnp.arange(pad_e, dtype=jnp.int32) % (NPAD - N))
    src = jnp.concatenate([edge_index[0], pad_idx]).reshape(NW, EPW)
    dst = jnp.concatenate([edge_index[1], pad_idx]).reshape(
        NW, NCHUNK, CHUNK)
    xp = jnp.pad(x, ((0, NPAD - N), (0, 0)))
    zeros = jnp.zeros((NPAD, DIM), jnp.float32)
    eye = jnp.eye(DIM, dtype=jnp.float32)
    bpad = jnp.pad(batch, (0, NPAD - N), constant_values=NG).reshape(1, NPAD)

    was = [eye] + [Wa[i] for i in range(4)]
    bas = [b1a.reshape(1, DIM)] + [Ba[i].reshape(1, DIM) for i in range(4)]
    wbs = [w1b] + [Wb[i] for i in range(4)]
    bbs = [b1b.reshape(1, DIM)] + [Bb[i].reshape(1, DIM) for i in range(4)]

    z = _proj(xp, w1a)
    for i in range(4):
        a = _edge_sum(zeros, z, src, dst)
        z = _mlp(a, z, was[i], bas[i], wbs[i], bbs[i],
                 gamma[i].reshape(1, DIM), beta[i].reshape(1, DIM))
    a = _edge_sum(zeros, z, src, dst)
    return _head(a, z, was[4], bas[4], wbs[4], bbs[4],
                 gamma[4].reshape(1, DIM), beta[4].reshape(1, DIM),
                 bpad, fc1_w, fc1_b.reshape(1, DIM),
                 fc2_w, fc2_b.reshape(1, NCLS))


# speedup vs baseline: 1.1164x; 1.0037x over previous
import functools

import jax
import jax.numpy as jnp
from jax import lax
from jax.experimental import pallas as pl
from jax.experimental.pallas import tpu as pltpu
from jax.experimental.pallas import tpu_sc as plsc

N = 10000
E = 320000
F_IN = 128
DIM = 32
NCLS = 10
NG = 64

NCORE = 2
NSUB = 16
NW = NCORE * NSUB
CHUNK = 1024
NPAD = 10112
RPT = NPAD // NSUB

INV_STD = 1.0 / (1.0 + 1e-5) ** 0.5


SUBK = 1
BLKE = SUBK * CHUNK
NBLK = -(-E // (NW * BLKE))
NCHUNK = NBLK * SUBK
EPW = NBLK * BLKE
EPAD = NW * EPW


def _edge_sum_body(zeros_hbm, table_hbm, srcr_hbm, dstr_hbm, out_hbm,
                   src_v, dst_v, rows0_v, rows1_v, acc_sh, gsem0, gsem1):
    cid = lax.axis_index("c")
    sid = lax.axis_index("s")
    w = cid * NSUB + sid
    pltpu.sync_copy(zeros_hbm.at[pl.ds(sid * RPT, RPT)],
                    acc_sh.at[pl.ds(sid * RPT, RPT)])
    pltpu.sync_copy(srcr_hbm.at[w], src_v)
    pltpu.sync_copy(dstr_hbm.at[w], dst_v)
    plsc.subcore_barrier()

    bufs = (rows0_v, rows1_v)
    sems = (gsem0, gsem1)

    def _gather(j, b):
        return pltpu.async_copy(
            table_hbm.at[src_v.at[pl.ds(j * BLKE, BLKE)]], bufs[b], sems[b])

    g = _gather(0, 0)
    for j in range(NBLK):
        g.wait()
        if j + 1 < NBLK:
            g = _gather(j + 1, (j + 1) % 2)
        for b in range(SUBK):
            pltpu.sync_copy(bufs[j % 2].at[pl.ds(b * CHUNK, CHUNK)],
                            acc_sh.at[dst_v.at[j * SUBK + b]], add=True)

    plsc.subcore_barrier()
    pltpu.sync_copy(acc_sh.at[pl.ds(sid * RPT, RPT)],
                    out_hbm.at[cid, pl.ds(sid * RPT, RPT)])


_edge_sum = pl.kernel(
    _edge_sum_body,
    out_type=jax.ShapeDtypeStruct((NCORE, NPAD, DIM), jnp.float32),
    mesh=plsc.VectorSubcoreMesh(core_axis_name="c", subcore_axis_name="s"),
    scratch_types=[
        pltpu.VMEM((EPW,), jnp.int32),
        pltpu.VMEM((NCHUNK, CHUNK), jnp.int32),
        pltpu.VMEM((BLKE, DIM), jnp.float32),
        pltpu.VMEM((BLKE, DIM), jnp.float32),
        pltpu.VMEM_SHARED((NPAD, DIM), jnp.float32),
        pltpu.SemaphoreType.DMA,
        pltpu.SemaphoreType.DMA,
    ],
    compiler_params=pltpu.CompilerParams(use_tc_tiling_on_sc=False),
)


def _proj_body(x_ref, w_ref, o_ref):
    o_ref[...] = jnp.dot(x_ref[...], w_ref[...],
                         preferred_element_type=jnp.float32)


def _proj(x, w):
    return pl.pallas_call(
        _proj_body,
        out_shape=jax.ShapeDtypeStruct((NPAD, DIM), jnp.float32),
    )(x, w)


def _mlp_body(a_ref, z_ref, wa_ref, ba_ref, wb_ref, bb_ref, g_ref, bt_ref,
              o_ref):
    agg = z_ref[...] + a_ref[0] + a_ref[1]
    t = jnp.dot(agg, wa_ref[...], preferred_element_type=jnp.float32)
    t = jnp.maximum(t + ba_ref[...], 0.0)
    t = jnp.dot(t, wb_ref[...], preferred_element_type=jnp.float32)
    t = jnp.maximum(t + bb_ref[...], 0.0)
    o_ref[...] = g_ref[...] * (t * INV_STD) + bt_ref[...]


def _mlp(a, z, wa, ba, wb, bb, g, bt):
    return pl.pallas_call(
        _mlp_body,
        out_shape=jax.ShapeDtypeStruct((NPAD, DIM), jnp.float32),
    )(a, z, wa, ba, wb, bb, g, bt)


def _head_body(a_ref, z_ref, wa_ref, ba_ref, wb_ref, bb_ref, g_ref, bt_ref,
               b_ref, w1_ref, b1_ref, w2_ref, b2_ref, o_ref):
    agg = z_ref[...] + a_ref[0] + a_ref[1]
    t = jnp.dot(agg, wa_ref[...], preferred_element_type=jnp.float32)
    t = jnp.maximum(t + ba_ref[...], 0.0)
    t = jnp.dot(t, wb_ref[...], preferred_element_type=jnp.float32)
    t = jnp.maximum(t + bb_ref[...], 0.0)
    h = g_ref[...] * (t * INV_STD) + bt_ref[...]
    gids = lax.broadcasted_iota(jnp.int32, (NG, NPAD), 0)
    m = (gids == b_ref[...]).astype(jnp.float32)
    pooled = jnp.dot(m, h, preferred_element_type=jnp.float32)
    t = jnp.dot(pooled, w1_ref[...], preferred_element_type=jnp.float32)
    t = jnp.maximum(t + b1_ref[...], 0.0)
    logits = jnp.dot(t, w2_ref[...], preferred_element_type=jnp.float32)
    logits = logits + b2_ref[...]
    mx = jnp.max(logits, axis=-1, keepdims=True)
    lse = jnp.log(jnp.sum(jnp.exp(logits - mx), axis=-1, keepdims=True)) + mx
    o_ref[...] = logits - lse


def _head(a, z, wa, ba, wb, bb, g, bt, b, w1, b1, w2, b2):
    return pl.pallas_call(
        _head_body,
        out_shape=jax.ShapeDtypeStruct((NG, NCLS), jnp.float32),
    )(a, z, wa, ba, wb, bb, g, bt, b, w1, b1, w2, b2)


def kernel(x, edge_index, batch, w1a, b1a, w1b, b1b, Wa, Ba, Wb, Bb,
           gamma, beta, fc1_w, fc1_b, fc2_w, fc2_b):
    pad_e = EPAD - E
    pad_idx = N + (jnp.arange(pad_e, dtype=jnp.int32) % (NPAD - N))
    src = jnp.concatenate([edge_index[0], pad_idx]).reshape(NW, EPW)
    dst = jnp.concatenate([edge_index[1], pad_idx]).reshape(
        NW, NCHUNK, CHUNK)
    xp = jnp.pad(x, ((0, NPAD - N), (0, 0)))
    zeros = jnp.zeros((NPAD, DIM), jnp.float32)
    eye = jnp.eye(DIM, dtype=jnp.float32)
    bpad = jnp.pad(batch, (0, NPAD - N), constant_values=NG).reshape(1, NPAD)

    was = [eye] + [Wa[i] for i in range(4)]
    bas = [b1a.reshape(1, DIM)] + [Ba[i].reshape(1, DIM) for i in range(4)]
    wbs = [w1b] + [Wb[i] for i in range(4)]
    bbs = [b1b.reshape(1, DIM)] + [Bb[i].reshape(1, DIM) for i in range(4)]

    z = _proj(xp, w1a)
    for i in range(4):
        a = _edge_sum(zeros, z, src, dst)
        z = _mlp(a, z, was[i], bas[i], wbs[i], bbs[i],
                 gamma[i].reshape(1, DIM), beta[i].reshape(1, DIM))
    a = _edge_sum(zeros, z, src, dst)
    return _head(a, z, was[4], bas[4], wbs[4], bbs[4],
                 gamma[4].reshape(1, DIM), beta[4].reshape(1, DIM),
                 bpad, fc1_w, fc1_b.reshape(1, DIM),
                 fc2_w, fc2_b.reshape(1, NCLS))
